# baseline (speedup 1.0000x reference)
import functools

import jax
import jax.numpy as jnp
from jax import lax
from jax.experimental import pallas as pl
from jax.experimental.pallas import tpu as pltpu
from jax.experimental.pallas import tpu_sc as plsc

_NC = 2
_NS = 16
_NW = _NC * _NS

_GSZ = 128
_NG = 4
_CH = _GSZ * _NG
_NB = 2


@functools.cache
def _build(V, D, N):
    assert N % (_NW * _CH * _NB) == 0
    npw = N // _NW
    ncyc = npw // (_CH * _NB)

    mesh = plsc.VectorSubcoreMesh(core_axis_name="c", subcore_axis_name="s")

    @functools.partial(
        pl.kernel,
        mesh=mesh,
        out_type=jax.ShapeDtypeStruct((N, D), jnp.float32),
        scratch_types=[
            pltpu.VMEM((npw,), jnp.int32),
            pltpu.VMEM((_NB, _CH, D), jnp.float32),
            pltpu.SemaphoreType.DMA,
            pltpu.SemaphoreType.DMA,
        ],
        compiler_params=pltpu.CompilerParams(use_tc_tiling_on_sc=False),
    )
    def gather_kernel(table_hbm, idx_hbm, out_hbm, idx_v, rows_v, gsem, ssem):
        wid = lax.axis_index("s") * _NC + lax.axis_index("c")
        base = wid * npw
        pltpu.sync_copy(idx_hbm.at[pl.ds(base, npw)], idx_v)

        def cycle(i, carry):
            for b in range(_NB):
                loc = (i * _NB + b) * _CH
                handles = []
                for g in range(_NG):
                    h = pltpu.async_copy(
                        table_hbm.at[idx_v.at[pl.ds(loc + g * _GSZ, _GSZ)]],
                        rows_v.at[b, pl.ds(g * _GSZ, _GSZ)],
                        gsem,
                    )
                    handles.append(h)
                for h in handles:
                    h.wait()
                pltpu.async_copy(
                    rows_v.at[b],
                    out_hbm.at[pl.ds(base + loc, _CH)],
                    ssem,
                ).wait()
            return carry

        lax.fori_loop(0, ncyc, cycle, 0)

    return gather_kernel


@jax.jit
def _run(token_ids, weight):
    B, H = token_ids.shape
    V, D = weight.shape
    N = B * H
    idx = token_ids.reshape(N).astype(jnp.int32)
    out = _build(V, D, N)(weight, idx)
    return out.reshape(B, H, D)


def kernel(token_ids, weight):
    return _run(token_ids, weight)


# speedup vs baseline: 1.0189x; 1.0189x over previous
import functools

import jax
import jax.numpy as jnp
from jax import lax
from jax.experimental import pallas as pl
from jax.experimental.pallas import tpu as pltpu
from jax.experimental.pallas import tpu_sc as plsc

_NC = 2
_NS = 16
_NW = _NC * _NS

_GSZ = 128
_NG = 4
_CH = _GSZ * _NG
_NB = 2


@functools.cache
def _build(V, D, N):
    assert N % (_NW * _CH * _NB) == 0
    npw = N // _NW
    ncyc = npw // (_CH * _NB)

    mesh = plsc.VectorSubcoreMesh(core_axis_name="c", subcore_axis_name="s")

    @functools.partial(
        pl.kernel,
        mesh=mesh,
        out_type=jax.ShapeDtypeStruct((N, D), jnp.float32),
        scratch_types=[
            pltpu.VMEM((npw,), jnp.int32),
            pltpu.VMEM((_NB, _CH, D), jnp.float32),
            pltpu.SemaphoreType.DMA,
            pltpu.SemaphoreType.DMA,
            pltpu.SemaphoreType.DMA,
            pltpu.SemaphoreType.DMA,
        ],
        compiler_params=pltpu.CompilerParams(use_tc_tiling_on_sc=False),
    )
    def gather_kernel(table_hbm, idx_hbm, out_hbm, idx_v, rows_v,
                      gsem0, gsem1, ssem0, ssem1):
        gsems = (gsem0, gsem1)
        ssems = (ssem0, ssem1)
        wid = lax.axis_index("s") * _NC + lax.axis_index("c")
        base = wid * npw
        pltpu.sync_copy(idx_hbm.at[pl.ds(base, npw)], idx_v)

        def fire_gathers(blk, b):
            loc = blk * _CH
            for g in range(_NG):
                pltpu.async_copy(
                    table_hbm.at[idx_v.at[pl.ds(loc + g * _GSZ, _GSZ)]],
                    rows_v.at[b, pl.ds(g * _GSZ, _GSZ)],
                    gsems[b],
                )

        def wait_gathers(b):
            for g in range(_NG):
                pltpu.make_async_copy(
                    table_hbm.at[idx_v.at[pl.ds(g * _GSZ, _GSZ)]],
                    rows_v.at[b, pl.ds(g * _GSZ, _GSZ)],
                    gsems[b],
                ).wait()

        def fire_store(blk, b):
            pltpu.async_copy(
                rows_v.at[b],
                out_hbm.at[pl.ds(base + blk * _CH, _CH)],
                ssems[b],
            )

        def wait_store(b):
            pltpu.make_async_copy(
                rows_v.at[b],
                out_hbm.at[pl.ds(base, _CH)],
                ssems[b],
            ).wait()

        for b in range(_NB):
            fire_gathers(b, b)

        def cycle(i, carry):
            for b in range(_NB):
                wait_gathers(b)
                fire_store(i * _NB + b, b)
            for b in range(_NB):
                wait_store(b)
                fire_gathers((i + 1) * _NB + b, b)
            return carry

        ngrp = ncyc - 1
        lax.fori_loop(0, ngrp, cycle, 0)

        last = ngrp * _NB
        for b in range(_NB):
            wait_gathers(b)
            fire_store(last + b, b)
        for b in range(_NB):
            wait_store(b)

    return gather_kernel


@jax.jit
def _run(token_ids, weight):
    B, H = token_ids.shape
    V, D = weight.shape
    N = B * H
    idx = token_ids.reshape(N).astype(jnp.int32)
    out = _build(V, D, N)(weight, idx)
    return out.reshape(B, H, D)


def kernel(token_ids, weight):
    return _run(token_ids, weight)
